# Initial kernel scaffold; baseline (speedup 1.0000x reference)
import jax
import jax.numpy as jnp
from jax.experimental import pallas as pl


def kernel(node_feats, edge_index, edge_feats, W, b, We, be):
    raise NotImplementedError("write your pallas kernel here")


# speedup vs baseline: 1.6767x; 1.6767x over previous
import functools

import jax
import jax.numpy as jnp
from jax import lax
from jax.experimental import pallas as pl
from jax.experimental.pallas import tpu as pltpu
from jax.experimental.pallas import tpu_sc as plsc

N_NODES = 10000
N_EDGES = 160000
D = 256
DH = 128
NC = 2
NS = 16
L = 16
ALPHA = 0.1

K = 80
EDGES_PER_TILE = N_EDGES // NS
BLOCKS = EDGES_PER_TILE // K
ROWS_A = 632
ROWS_B = N_NODES - 15 * ROWS_A


def _select_row(b2d, c):
    mask = lax.broadcasted_iota(jnp.int32, b2d.shape, 0) == c
    return jnp.sum(jnp.where(mask, b2d, 0.0), axis=0, keepdims=True)


def _h_body(node_ref, w_ref, b_ref, out_ref):
    bias = _select_row(b_ref[...], pl.program_id(0))
    out_ref[...] = lax.dot_general(
        node_ref[...], w_ref[...], (((1,), (1,)), ((), ())),
        preferred_element_type=jnp.float32) + bias


def _f_body(ef_ref, we_ref, be_ref, out_ref):
    bias = _select_row(be_ref[...], pl.program_id(0))
    out_ref[...] = lax.dot_general(
        ef_ref[...], we_ref[...], (((1,), (1,)), ((), ())),
        preferred_element_type=jnp.float32) + bias


def _fin_body(node_ref, agg_ref, out_ref):
    out_ref[...] = node_ref[...] + ALPHA * jnp.maximum(agg_ref[...], 0.0)


_SC_MESH = plsc.VectorSubcoreMesh(core_axis_name="c", subcore_axis_name="s")


@functools.partial(
    pl.kernel,
    out_type=jax.ShapeDtypeStruct((NC * N_NODES, DH), jnp.float32),
    mesh=_SC_MESH,
    scratch_types=[
        pltpu.VMEM((K,), jnp.int32),
        pltpu.VMEM((K,), jnp.int32),
        pltpu.VMEM((K, DH), jnp.float32),
        pltpu.VMEM((K, DH), jnp.float32),
        pltpu.VMEM_SHARED((N_NODES, DH), jnp.float32),
        pltpu.SemaphoreType.DMA,
    ],
)
def _sc_edge_aggregate(src2_hbm, dst_hbm, h2_hbm, f2_hbm, agg2_hbm,
                       src_v, dst_v, h_v, f_v, agg_sh, gsem):
    cid = lax.axis_index("c")
    sid = lax.axis_index("s")

    zero = jnp.zeros((L,), jnp.float32)

    def zrow(r, _):
        for j in range(DH // L):
            h_v[r, pl.ds(j * L, L)] = zero
        return 0

    lax.fori_loop(0, K, zrow, 0)
    row0 = sid * ROWS_A

    def _zfill(nrows):
        nfull, rem = nrows // K, nrows % K
        for i in range(nfull):
            pltpu.sync_copy(h_v, agg_sh.at[pl.ds(row0 + i * K, K)])
        if rem:
            pltpu.sync_copy(h_v.at[pl.ds(0, rem)],
                            agg_sh.at[pl.ds(row0 + nfull * K, rem)])

    @pl.when(sid < NS - 1)
    def _():
        _zfill(ROWS_A)

    @pl.when(sid == NS - 1)
    def _():
        _zfill(ROWS_B)

    plsc.subcore_barrier()

    e_tile = sid * EDGES_PER_TILE

    def body(g, _):
        e_base = e_tile + g * K
        pltpu.sync_copy(src2_hbm.at[pl.ds(cid * N_EDGES + e_base, K)], src_v)
        pltpu.sync_copy(dst_hbm.at[pl.ds(e_base, K)], dst_v)
        pltpu.async_copy(h2_hbm.at[src_v], h_v, gsem).wait()
        pltpu.sync_copy(f2_hbm.at[pl.ds(cid * N_EDGES + e_base, K)], f_v)

        def mrow(r, _):
            for j in range(DH // L):
                h_v[r, pl.ds(j * L, L)] = (h_v[r, pl.ds(j * L, L)]
                                           * f_v[r, pl.ds(j * L, L)])
            return 0

        lax.fori_loop(0, K, mrow, 0)
        pltpu.sync_copy(h_v, agg_sh.at[dst_v], add=True)
        return 0

    lax.fori_loop(0, BLOCKS, body, 0)
    plsc.subcore_barrier()

    @pl.when(sid < NS - 1)
    def _():
        pltpu.sync_copy(agg_sh.at[pl.ds(row0, ROWS_A)],
                        agg2_hbm.at[pl.ds(cid * N_NODES + row0, ROWS_A)])

    @pl.when(sid == NS - 1)
    def _():
        pltpu.sync_copy(agg_sh.at[pl.ds(row0, ROWS_B)],
                        agg2_hbm.at[pl.ds(cid * N_NODES + row0, ROWS_B)])


def kernel(node_feats, edge_index, edge_feats, W, b, We, be):
    src = edge_index[0].astype(jnp.int32)
    dst = edge_index[1].astype(jnp.int32)
    src2 = jnp.concatenate([src, src + N_NODES])
    b2 = b.reshape(NC, DH)
    be2 = be.reshape(NC, DH)

    nb_h = 10
    bh = N_NODES // nb_h
    h2 = pl.pallas_call(
        _h_body,
        grid=(NC, nb_h),
        in_specs=[
            pl.BlockSpec((bh, D), lambda c, i: (i, 0)),
            pl.BlockSpec((DH, D), lambda c, i: (c, 0)),
            pl.BlockSpec((NC, DH), lambda c, i: (0, 0)),
        ],
        out_specs=pl.BlockSpec((bh, DH), lambda c, i: (c * nb_h + i, 0)),
        out_shape=jax.ShapeDtypeStruct((NC * N_NODES, DH), jnp.float32),
    )(node_feats, W, b2)

    nb_f = 80
    bf = N_EDGES // nb_f
    f2 = pl.pallas_call(
        _f_body,
        grid=(NC, nb_f),
        in_specs=[
            pl.BlockSpec((bf, 16), lambda c, j: (j, 0)),
            pl.BlockSpec((DH, 16), lambda c, j: (c, 0)),
            pl.BlockSpec((NC, DH), lambda c, j: (0, 0)),
        ],
        out_specs=pl.BlockSpec((bf, DH), lambda c, j: (c * nb_f + j, 0)),
        out_shape=jax.ShapeDtypeStruct((NC * N_EDGES, DH), jnp.float32),
    )(edge_feats, We, be2)

    agg2 = _sc_edge_aggregate(src2, dst, h2, f2)

    out = pl.pallas_call(
        _fin_body,
        grid=(NC, nb_h),
        in_specs=[
            pl.BlockSpec((bh, DH), lambda c, i: (i, c)),
            pl.BlockSpec((bh, DH), lambda c, i: (c * nb_h + i, 0)),
        ],
        out_specs=pl.BlockSpec((bh, DH), lambda c, i: (i, c)),
        out_shape=jax.ShapeDtypeStruct((N_NODES, D), jnp.float32),
    )(node_feats, agg2)
    return out


# speedup vs baseline: 2.6871x; 1.6027x over previous
import functools

import jax
import jax.numpy as jnp
from jax import lax
from jax.experimental import pallas as pl
from jax.experimental.pallas import tpu as pltpu
from jax.experimental.pallas import tpu_sc as plsc

N_NODES = 10000
N_EDGES = 160000
D = 256
DH = 128
NC = 2
NS = 16
L = 16
ALPHA = 0.1

K = 80
EDGES_PER_TILE = N_EDGES // NS
BLOCKS = EDGES_PER_TILE // K
ROWS_A = 632
ROWS_B = N_NODES - 15 * ROWS_A


def _select_row(b2d, c):
    mask = lax.broadcasted_iota(jnp.int32, b2d.shape, 0) == c
    return jnp.sum(jnp.where(mask, b2d, 0.0), axis=0, keepdims=True)


def _h_body(node_ref, w_ref, b_ref, out_ref):
    bias = _select_row(b_ref[...], pl.program_id(0))
    out_ref[...] = lax.dot_general(
        node_ref[...], w_ref[...], (((1,), (1,)), ((), ())),
        preferred_element_type=jnp.float32) + bias


def _f_body(ef_ref, we_ref, be_ref, out_ref):
    bias = _select_row(be_ref[...], pl.program_id(0))
    out_ref[...] = lax.dot_general(
        ef_ref[...], we_ref[...], (((1,), (1,)), ((), ())),
        preferred_element_type=jnp.float32) + bias


def _fin_body(node_ref, agg_ref, out_ref):
    out_ref[...] = node_ref[...] + ALPHA * jnp.maximum(agg_ref[...], 0.0)


_SC_MESH = plsc.VectorSubcoreMesh(core_axis_name="c", subcore_axis_name="s")


@functools.partial(
    pl.kernel,
    out_type=jax.ShapeDtypeStruct((NC * N_NODES, DH), jnp.float32),
    mesh=_SC_MESH,
    scratch_types=[
        pltpu.VMEM((2, K), jnp.int32),
        pltpu.VMEM((2, K), jnp.int32),
        pltpu.VMEM((2, K, DH), jnp.float32),
        pltpu.VMEM((2, K, DH), jnp.float32),
        pltpu.VMEM_SHARED((N_NODES, DH), jnp.float32),
        pltpu.SemaphoreType.DMA,
        pltpu.SemaphoreType.DMA,
        pltpu.SemaphoreType.DMA,
        pltpu.SemaphoreType.DMA,
        pltpu.SemaphoreType.DMA,
        pltpu.SemaphoreType.DMA,
    ],
)
def _sc_edge_aggregate(src2_hbm, dst_hbm, h2_hbm, f2_hbm, agg2_hbm,
                       src_v, dst_v, h_v, f_v, agg_sh,
                       si0, si1, sg0, sg1, sf0, sf1):
    cid = lax.axis_index("c")
    sid = lax.axis_index("s")
    sis, sgs, sfs = (si0, si1), (sg0, sg1), (sf0, sf1)

    zero = jnp.zeros((L,), jnp.float32)

    def zrow(r, _):
        for j in range(DH // L):
            h_v[0, r, pl.ds(j * L, L)] = zero
        return 0

    lax.fori_loop(0, K, zrow, 0)
    row0 = sid * ROWS_A

    def _zfill(nrows):
        nfull, rem = nrows // K, nrows % K
        for i in range(nfull):
            pltpu.sync_copy(h_v.at[0], agg_sh.at[pl.ds(row0 + i * K, K)])
        if rem:
            pltpu.sync_copy(h_v.at[0, pl.ds(0, rem)],
                            agg_sh.at[pl.ds(row0 + nfull * K, rem)])

    @pl.when(sid < NS - 1)
    def _():
        _zfill(ROWS_A)

    @pl.when(sid == NS - 1)
    def _():
        _zfill(ROWS_B)

    plsc.subcore_barrier()

    e_tile = sid * EDGES_PER_TILE

    def idx_copies(g, p):
        e_base = e_tile + g * K
        return (
            pltpu.make_async_copy(
                src2_hbm.at[pl.ds(cid * N_EDGES + e_base, K)],
                src_v.at[p], sis[p]),
            pltpu.make_async_copy(
                dst_hbm.at[pl.ds(e_base, K)], dst_v.at[p], sis[p]),
        )

    def gf_copies(g, p):
        e_base = e_tile + g * K
        return (
            pltpu.make_async_copy(h2_hbm.at[src_v.at[p]], h_v.at[p], sgs[p]),
            pltpu.make_async_copy(
                f2_hbm.at[pl.ds(cid * N_EDGES + e_base, K)],
                f_v.at[p], sfs[p]),
        )

    for c_ in idx_copies(0, 0):
        c_.start()
    for c_ in idx_copies(0, 0):
        c_.wait()
    for c_ in gf_copies(0, 0):
        c_.start()
    for c_ in idx_copies(1, 1):
        c_.start()

    def outer(step, _):
        g0 = step * 2
        for p in range(2):
            g = g0 + p

            @pl.when(g < BLOCKS)
            def _():
                @pl.when(g + 1 < BLOCKS)
                def _():
                    for c_ in idx_copies(g + 1, 1 - p):
                        c_.wait()
                    for c_ in gf_copies(g + 1, 1 - p):
                        c_.start()

                for c_ in gf_copies(g, p):
                    c_.wait()

                def mrow(r, _):
                    for j in range(DH // L):
                        h_v[p, r, pl.ds(j * L, L)] = (
                            h_v[p, r, pl.ds(j * L, L)]
                            * f_v[p, r, pl.ds(j * L, L)])
                    return 0

                lax.fori_loop(0, K, mrow, 0)
                pltpu.sync_copy(h_v.at[p], agg_sh.at[dst_v.at[p]], add=True)

                @pl.when(g + 2 < BLOCKS)
                def _():
                    for c_ in idx_copies(g + 2, p):
                        c_.start()

        return 0

    lax.fori_loop(0, (BLOCKS + 1) // 2, outer, 0)
    plsc.subcore_barrier()

    @pl.when(sid < NS - 1)
    def _():
        pltpu.sync_copy(agg_sh.at[pl.ds(row0, ROWS_A)],
                        agg2_hbm.at[pl.ds(cid * N_NODES + row0, ROWS_A)])

    @pl.when(sid == NS - 1)
    def _():
        pltpu.sync_copy(agg_sh.at[pl.ds(row0, ROWS_B)],
                        agg2_hbm.at[pl.ds(cid * N_NODES + row0, ROWS_B)])


def kernel(node_feats, edge_index, edge_feats, W, b, We, be):
    src = edge_index[0].astype(jnp.int32)
    dst = edge_index[1].astype(jnp.int32)
    src2 = jnp.concatenate([src, src + N_NODES])
    b2 = b.reshape(NC, DH)
    be2 = be.reshape(NC, DH)

    nb_h = 10
    bh = N_NODES // nb_h
    h2 = pl.pallas_call(
        _h_body,
        grid=(NC, nb_h),
        in_specs=[
            pl.BlockSpec((bh, D), lambda c, i: (i, 0)),
            pl.BlockSpec((DH, D), lambda c, i: (c, 0)),
            pl.BlockSpec((NC, DH), lambda c, i: (0, 0)),
        ],
        out_specs=pl.BlockSpec((bh, DH), lambda c, i: (c * nb_h + i, 0)),
        out_shape=jax.ShapeDtypeStruct((NC * N_NODES, DH), jnp.float32),
    )(node_feats, W, b2)

    nb_f = 80
    bf = N_EDGES // nb_f
    f2 = pl.pallas_call(
        _f_body,
        grid=(NC, nb_f),
        in_specs=[
            pl.BlockSpec((bf, 16), lambda c, j: (j, 0)),
            pl.BlockSpec((DH, 16), lambda c, j: (c, 0)),
            pl.BlockSpec((NC, DH), lambda c, j: (0, 0)),
        ],
        out_specs=pl.BlockSpec((bf, DH), lambda c, j: (c * nb_f + j, 0)),
        out_shape=jax.ShapeDtypeStruct((NC * N_EDGES, DH), jnp.float32),
    )(edge_feats, We, be2)

    agg2 = _sc_edge_aggregate(src2, dst, h2, f2)

    out = pl.pallas_call(
        _fin_body,
        grid=(NC, nb_h),
        in_specs=[
            pl.BlockSpec((bh, DH), lambda c, i: (i, c)),
            pl.BlockSpec((bh, DH), lambda c, i: (c * nb_h + i, 0)),
        ],
        out_specs=pl.BlockSpec((bh, DH), lambda c, i: (i, c)),
        out_shape=jax.ShapeDtypeStruct((N_NODES, D), jnp.float32),
    )(node_feats, agg2)
    return out
